# Initial kernel scaffold; baseline (speedup 1.0000x reference)
import jax
import jax.numpy as jnp
from jax.experimental import pallas as pl


def kernel(agent_obs, hideout_obs, timestep_obs, num_agents, W_i, Wh_i, bh_i, b_i, w_c_i, W_f, Wh_f, bh_f, b_f, w_c_f, W_c, Wh_c, bh_c, b_c, W_o, Wh_o, bh_o, b_o, w_c_o):
    raise NotImplementedError("write your pallas kernel here")


# speedup vs baseline: 1.5361x; 1.5361x over previous
import functools

import jax
import jax.numpy as jnp
from jax.experimental import pallas as pl

P = 4


def _lstm_body(x_ref, nrow_ref, cnt_ref, w_ref, wh_ref, b_ref, wc_ref, out_ref):
    G, T, RPB, _ = x_ref.shape
    DH4 = wc_ref.shape[1]
    R = G * RPB
    W = w_ref[...]
    Wh = wh_ref[...]
    bias = b_ref[...]
    wci = wc_ref[0:1, :]
    wcf = wc_ref[1:2, :]
    wco = wc_ref[2:3, :]

    def step(t, carry):
        H, C = carry
        xt = x_ref[:, t, :, :].reshape(R, x_ref.shape[3])
        g = (jnp.dot(xt, W, preferred_element_type=jnp.float32)
             + jnp.dot(H, Wh, preferred_element_type=jnp.float32)
             + bias)
        gi = g[:, 0:DH4]
        gf = g[:, DH4:2 * DH4]
        gc = g[:, 2 * DH4:3 * DH4]
        go = g[:, 3 * DH4:4 * DH4]
        I = jax.nn.sigmoid(gi + wci * C)
        F = jax.nn.sigmoid(gf + wcf * C)
        Tg = jnp.tanh(gc)
        C2 = F * C + I * Tg
        O = jax.nn.sigmoid(go + wco * C2)
        H2 = O * jnp.tanh(C2)
        return H2, C2

    z = jnp.zeros((R, DH4), jnp.float32)
    H, _ = jax.lax.fori_loop(0, T, step, (z, z))

    D_H = DH4 // P
    r_iota = jax.lax.broadcasted_iota(jnp.int32, (R, DH4), 0)
    l_iota = jax.lax.broadcasted_iota(jnp.int32, (R, DH4), 1)
    node = (r_iota % RPB) * P + l_iota // D_H
    Hm = H * (node < nrow_ref[...]).astype(jnp.float32)
    s = Hm.reshape(G, RPB, DH4).sum(axis=1)
    s = (s[:, 0:D_H] + s[:, D_H:2 * D_H]
         + s[:, 2 * D_H:3 * D_H] + s[:, 3 * D_H:4 * D_H])
    out_ref[...] = s / cnt_ref[...]


def kernel(agent_obs, hideout_obs, timestep_obs, num_agents,
           W_i, Wh_i, bh_i, b_i, w_c_i,
           W_f, Wh_f, bh_f, b_f, w_c_f,
           W_c, Wh_c, bh_c, b_c,
           W_o, Wh_o, bh_o, b_o, w_c_o):
    B, T, MAX_N, D_IN = agent_obs.shape
    D_H = W_i.shape[1]
    RPB = MAX_N // P
    G = 8

    x = agent_obs.reshape(B, T, RPB, P * D_IN)
    eye = jnp.eye(P, dtype=jnp.float32)
    Wbd = jnp.concatenate(
        [jnp.kron(eye, Wg) for Wg in (W_i, W_f, W_c, W_o)], axis=1)
    Whbd = jnp.concatenate(
        [jnp.kron(eye, Wg) for Wg in (Wh_i, Wh_f, Wh_c, Wh_o)], axis=1)
    bias = jnp.concatenate(
        [jnp.tile(bh + b.reshape(-1), P)
         for bh, b in ((bh_i, b_i), (bh_f, b_f), (bh_c, b_c), (bh_o, b_o))]
    ).reshape(1, 4 * P * D_H)
    wc = jnp.stack(
        [jnp.tile(w.reshape(-1), P) for w in (w_c_i, w_c_f, w_c_o)], axis=0)
    na_i32 = num_agents.astype(jnp.int32)
    na_row = jnp.repeat(na_i32, RPB).reshape(B * RPB, 1)
    counts = na_i32.astype(jnp.float32).reshape(B, 1)

    pooled = pl.pallas_call(
        _lstm_body,
        grid=(B // G,),
        in_specs=[
            pl.BlockSpec((G, T, RPB, P * D_IN), lambda i: (i, 0, 0, 0)),
            pl.BlockSpec((G * RPB, 1), lambda i: (i, 0)),
            pl.BlockSpec((G, 1), lambda i: (i, 0)),
            pl.BlockSpec((P * D_IN, 4 * P * D_H), lambda i: (0, 0)),
            pl.BlockSpec((P * D_H, 4 * P * D_H), lambda i: (0, 0)),
            pl.BlockSpec((1, 4 * P * D_H), lambda i: (0, 0)),
            pl.BlockSpec((3, P * D_H), lambda i: (0, 0)),
        ],
        out_specs=pl.BlockSpec((G, D_H), lambda i: (i, 0)),
        out_shape=jax.ShapeDtypeStruct((B, D_H), jnp.float32),
    )(x, na_row, counts, Wbd, Whbd, bias, wc)

    return jnp.concatenate([pooled, hideout_obs, timestep_obs], axis=-1)


# speedup vs baseline: 1.8247x; 1.1879x over previous
import functools

import jax
import jax.numpy as jnp
from jax.experimental import pallas as pl

P = 4


def _lstm_body(x_ref, nrow_ref, cnt_ref, w_ref, wh_ref, b_ref, wc_ref, out_ref):
    G, T, RPB, _ = x_ref.shape
    DH4 = wc_ref.shape[1]
    R = G * RPB
    W = w_ref[...]
    Wh = wh_ref[...]
    bias = b_ref[...]
    wci = wc_ref[0:1, :]
    wcf = wc_ref[1:2, :]
    wco = wc_ref[2:3, :]

    def step(t, carry):
        H, C = carry
        xt = x_ref[:, t, :, :].reshape(R, x_ref.shape[3])
        g = (jnp.dot(xt, W, preferred_element_type=jnp.float32)
             + jnp.dot(H, Wh, preferred_element_type=jnp.float32)
             + bias)
        gi = g[:, 0:DH4]
        gf = g[:, DH4:2 * DH4]
        gc = g[:, 2 * DH4:3 * DH4]
        go = g[:, 3 * DH4:4 * DH4]
        I = 0.5 * jnp.tanh(gi + wci * C) + 0.5
        F = 0.5 * jnp.tanh(gf + wcf * C) + 0.5
        Tg = jnp.tanh(gc)
        C2 = F * C + I * Tg
        O = 0.5 * jnp.tanh(go + wco * C2) + 0.5
        H2 = O * jnp.tanh(C2)
        return H2, C2

    z = jnp.zeros((R, DH4), jnp.float32)
    H, _ = jax.lax.fori_loop(0, T, step, (z, z), unroll=2)

    D_H = DH4 // P
    r_iota = jax.lax.broadcasted_iota(jnp.int32, (R, DH4), 0)
    l_iota = jax.lax.broadcasted_iota(jnp.int32, (R, DH4), 1)
    node = (r_iota % RPB) * P + l_iota // D_H
    Hm = H * (node < nrow_ref[...]).astype(jnp.float32)
    s = Hm.reshape(G, RPB, DH4).sum(axis=1)
    s = (s[:, 0:D_H] + s[:, D_H:2 * D_H]
         + s[:, 2 * D_H:3 * D_H] + s[:, 3 * D_H:4 * D_H])
    out_ref[...] = s / cnt_ref[...]


def kernel(agent_obs, hideout_obs, timestep_obs, num_agents,
           W_i, Wh_i, bh_i, b_i, w_c_i,
           W_f, Wh_f, bh_f, b_f, w_c_f,
           W_c, Wh_c, bh_c, b_c,
           W_o, Wh_o, bh_o, b_o, w_c_o):
    B, T, MAX_N, D_IN = agent_obs.shape
    D_H = W_i.shape[1]
    RPB = MAX_N // P
    G = 8

    x = agent_obs.reshape(B, T, RPB, P * D_IN)
    eye = jnp.eye(P, dtype=jnp.float32)
    gate_s = (0.5, 0.5, 1.0, 0.5)
    Wbd = jnp.concatenate(
        [s * jnp.kron(eye, Wg)
         for s, Wg in zip(gate_s, (W_i, W_f, W_c, W_o))], axis=1)
    Whbd = jnp.concatenate(
        [s * jnp.kron(eye, Wg)
         for s, Wg in zip(gate_s, (Wh_i, Wh_f, Wh_c, Wh_o))], axis=1)
    bias = jnp.concatenate(
        [s * jnp.tile(bh + b.reshape(-1), P)
         for s, (bh, b) in zip(gate_s, ((bh_i, b_i), (bh_f, b_f),
                                        (bh_c, b_c), (bh_o, b_o)))]
    ).reshape(1, 4 * P * D_H)
    wc = jnp.stack(
        [0.5 * jnp.tile(w.reshape(-1), P) for w in (w_c_i, w_c_f, w_c_o)],
        axis=0)
    na_i32 = num_agents.astype(jnp.int32)
    na_row = jnp.repeat(na_i32, RPB).reshape(B * RPB, 1)
    counts = na_i32.astype(jnp.float32).reshape(B, 1)

    pooled = pl.pallas_call(
        _lstm_body,
        grid=(B // G,),
        in_specs=[
            pl.BlockSpec((G, T, RPB, P * D_IN), lambda i: (i, 0, 0, 0)),
            pl.BlockSpec((G * RPB, 1), lambda i: (i, 0)),
            pl.BlockSpec((G, 1), lambda i: (i, 0)),
            pl.BlockSpec((P * D_IN, 4 * P * D_H), lambda i: (0, 0)),
            pl.BlockSpec((P * D_H, 4 * P * D_H), lambda i: (0, 0)),
            pl.BlockSpec((1, 4 * P * D_H), lambda i: (0, 0)),
            pl.BlockSpec((3, P * D_H), lambda i: (0, 0)),
        ],
        out_specs=pl.BlockSpec((G, D_H), lambda i: (i, 0)),
        out_shape=jax.ShapeDtypeStruct((B, D_H), jnp.float32),
    )(x, na_row, counts, Wbd, Whbd, bias, wc)

    return jnp.concatenate([pooled, hideout_obs, timestep_obs], axis=-1)
